# baseline (speedup 1.0000x reference)
import jax
import jax.numpy as jnp
from jax import lax
from jax.experimental import pallas as pl
from jax.experimental.pallas import tpu as pltpu
from jax.experimental.pallas import tpu_sc as plsc

N = 100000
E = 1600000
H = 12
LANES = 128
NROWS = 800
NPAD = NROWS * LANES
NSUB = 16
NCORE = 2
NW = NSUB * NCORE
EPW = E // NW
NCH = 25
CH = EPW // NCH
NBLK = E // LANES
BPW = NBLK // NW
NCH_H = 26
BPC = BPW // NCH_H
CH_H = BPC * LANES
SL = NPAD // NSUB
GRID = 4
BR = NROWS // GRID

_sc_mesh = plsc.VectorSubcoreMesh(core_axis_name="c", subcore_axis_name="s")


def _zero_fill(ref, n):
    def body(i, carry):
        ref[pl.ds(i * 16, 16)] = jnp.zeros((16,), jnp.float32)
        return carry
    lax.fori_loop(0, n // 16, body, 0)


def _ones_fill(ref, n):
    def body(i, carry):
        ref[pl.ds(i * 16, 16)] = jnp.ones((16,), jnp.float32)
        return carry
    lax.fori_loop(0, n // 16, body, 0)


def _sc_hist_body(ei_h, out_h, flat_h, ibuf0, ibuf1, ibuf2,
                  didx0, didx1, didx2, vals, obuf, acc,
                  sem_i0, sem_i1, sem_i2, sem_s0, sem_s1, sem_s2,
                  sem_w0, sem_w1, sem_w2):
    c = lax.axis_index("c")
    s = lax.axis_index("s")
    w = c * NSUB + s
    _zero_fill(obuf, SL)
    _ones_fill(vals, CH_H)
    pltpu.sync_copy(obuf, acc.at[pl.ds(s * SL, SL)])
    plsc.subcore_barrier()
    base = w * BPW * LANES

    ibuf = [ibuf0, ibuf1, ibuf2]
    didx = [didx0, didx1, didx2]
    semi = [sem_i0, sem_i1, sem_i2]
    sems = [sem_s0, sem_s1, sem_s2]
    semw = [sem_w0, sem_w1, sem_w2]
    idesc = [None, None, None]
    sdesc = [None, None, None]
    wdesc = [None, None, None]

    def issue_idx(j):
        b = j % 3
        off = base + j * CH_H
        idesc[b] = pltpu.async_copy(
            ei_h.at[pl.ds(0, 2), pl.ds(off, CH_H)], ibuf[b], semi[b])

    issue_idx(0)
    issue_idx(1)
    for j in range(NCH_H):
        b = j % 3
        idesc[b].wait()
        idesc[b] = None
        off = base + j * CH_H

        @plsc.parallel_loop(0, CH_H // 16, unroll=5)
        def cp(i, _ib=ibuf[b], _db=didx[b]):
            _db[pl.ds(i * 16, 16)] = _ib[1, pl.ds(i * 16, 16)]

        sdesc[b] = pltpu.async_copy(vals, acc.at[didx[b]], sems[b],
                                    add=True)
        wdesc[b] = (
            pltpu.async_copy(ibuf[b].at[0], flat_h.at[pl.ds(off, CH_H)],
                             semw[b]),
            pltpu.async_copy(didx[b], flat_h.at[pl.ds(E + off, CH_H)],
                             semw[b]),
        )
        if j + 2 < NCH_H:
            nb = (j + 2) % 3
            if sdesc[nb] is not None:
                sdesc[nb].wait()
                sdesc[nb] = None
                for d in wdesc[nb]:
                    d.wait()
                wdesc[nb] = None
            issue_idx(j + 2)
    for b in range(3):
        if sdesc[b] is not None:
            sdesc[b].wait()
            for d in wdesc[b]:
                d.wait()

    @pl.when(w < 20)
    def _tail():
        off = (NW * BPW + w) * LANES
        pltpu.sync_copy(ei_h.at[pl.ds(0, 2), pl.ds(off, LANES)],
                        ibuf0.at[:, pl.ds(0, LANES)])

        @plsc.parallel_loop(0, LANES // 16, unroll=4)
        def cpt(i):
            didx0[pl.ds(i * 16, 16)] = ibuf0[1, pl.ds(i * 16, 16)]
            didx1[pl.ds(i * 16, 16)] = ibuf0[0, pl.ds(i * 16, 16)]

        pltpu.sync_copy(vals.at[pl.ds(0, LANES)],
                        acc.at[didx0.at[pl.ds(0, LANES)]], add=True)
        pltpu.sync_copy(didx1.at[pl.ds(0, LANES)],
                        flat_h.at[pl.ds(off, LANES)])
        pltpu.sync_copy(didx0.at[pl.ds(0, LANES)],
                        flat_h.at[pl.ds(E + off, LANES)])

    plsc.subcore_barrier()
    pltpu.sync_copy(acc.at[pl.ds(s * SL, SL)], obuf)
    pltpu.sync_copy(obuf, out_h.at[c, s])


_sc_hist = pl.kernel(
    _sc_hist_body,
    out_type=[
        jax.ShapeDtypeStruct((NCORE, NSUB, SL), jnp.float32),
        jax.ShapeDtypeStruct((2 * E,), jnp.int32),
    ],
    mesh=_sc_mesh,
    scratch_types=(
        [pltpu.VMEM((2, CH_H), jnp.int32) for _ in range(3)]
        + [pltpu.VMEM((CH_H,), jnp.int32) for _ in range(3)]
        + [
            pltpu.VMEM((CH_H,), jnp.float32),
            pltpu.VMEM((SL,), jnp.float32),
            pltpu.VMEM_SHARED((NPAD,), jnp.float32),
        ]
        + [pltpu.SemaphoreType.DMA for _ in range(9)]
    ),
)


def _sc_spmv_body(ei_h, vec_h, out_h, vtab,
                  sidx0, sidx1, sidx2, didx0, didx1, didx2,
                  vals0, vals1, vals2, acc,
                  sem_v, sem_i0, sem_i1, sem_i2, sem_s0, sem_s1, sem_s2):
    c = lax.axis_index("c")
    s = lax.axis_index("s")
    w = c * NSUB + s
    vdesc = pltpu.async_copy(vec_h.at[pl.ds(0, N)], vtab, sem_v)
    _zero_fill(vals0, CH)
    _zero_fill(vals1, CH)
    _zero_fill(vals2, CH)
    pltpu.sync_copy(vals0, acc.at[pl.ds(s * SL, CH)])
    pltpu.sync_copy(vals1, acc.at[pl.ds(s * SL + CH, CH)])
    pltpu.sync_copy(vals2, acc.at[pl.ds(s * SL + 2 * CH, CH)])
    pltpu.sync_copy(vals0.at[pl.ds(0, SL - 3 * CH)],
                    acc.at[pl.ds(s * SL + 3 * CH, SL - 3 * CH)])
    base = w * EPW

    sidx = [sidx0, sidx1, sidx2]
    didx = [didx0, didx1, didx2]
    vals = [vals0, vals1, vals2]
    semi = [sem_i0, sem_i1, sem_i2]
    sems = [sem_s0, sem_s1, sem_s2]
    idesc = [None, None, None]
    sdesc = [None, None, None]

    def issue_idx(j):
        b = j % 3
        off = base + j * CH
        idesc[b] = (
            pltpu.async_copy(ei_h.at[pl.ds(off, CH)], sidx[b], semi[b]),
            pltpu.async_copy(ei_h.at[pl.ds(E + off, CH)], didx[b], semi[b]),
        )

    issue_idx(0)
    issue_idx(1)
    vdesc.wait()
    plsc.subcore_barrier()
    for j in range(NCH):
        b = j % 3
        for d in idesc[b]:
            d.wait()
        idesc[b] = None

        @plsc.parallel_loop(0, CH // 16, unroll=5)
        def grp(i, _sb=sidx[b], _vb=vals[b]):
            si = _sb[pl.ds(i * 16, 16)]
            _vb[pl.ds(i * 16, 16)] = plsc.load_gather(vtab, [si])

        sdesc[b] = pltpu.async_copy(vals[b], acc.at[didx[b]], sems[b], add=True)
        if j + 2 < NCH:
            nb = (j + 2) % 3
            if sdesc[nb] is not None:
                sdesc[nb].wait()
                sdesc[nb] = None
            issue_idx(j + 2)
    for b in range(3):
        if sdesc[b] is not None:
            sdesc[b].wait()
    plsc.subcore_barrier()
    obase = c * NPAD + s * SL
    for k in range(3):
        pltpu.sync_copy(acc.at[pl.ds(s * SL + k * CH, CH)], vals[k])
        pltpu.sync_copy(vals[k], out_h.at[pl.ds(obase + k * CH, CH)])
    tail = SL - 3 * CH
    pltpu.sync_copy(acc.at[pl.ds(s * SL + 3 * CH, tail)], vals0.at[pl.ds(0, tail)])
    pltpu.sync_copy(vals0.at[pl.ds(0, tail)], out_h.at[pl.ds(obase + 3 * CH, tail)])


_sc_spmv = pl.kernel(
    _sc_spmv_body,
    out_type=jax.ShapeDtypeStruct((NCORE * NPAD,), jnp.float32),
    mesh=_sc_mesh,
    compiler_params=pltpu.CompilerParams(needs_layout_passes=False),
    scratch_types=(
        [pltpu.VMEM((N,), jnp.float32)]
        + [pltpu.VMEM((CH,), jnp.int32) for _ in range(6)]
        + [pltpu.VMEM((CH,), jnp.float32) for _ in range(3)]
        + [pltpu.VMEM_SHARED((NPAD,), jnp.float32)]
        + [pltpu.SemaphoreType.DMA for _ in range(7)]
    ),
)


def _tc_pre_body(xT_ref, W1_ref, W2_ref, Wl_ref, degp_ref, dinv_ref, zhat_ref):
    v = jnp.dot(W1_ref[...], jnp.dot(W2_ref[...], Wl_ref[...]),
                preferred_element_type=jnp.float32)
    vb = jnp.broadcast_to(v, (H, LANES))
    z = jnp.zeros((BR, LANES), jnp.float32)
    for j in range(H):
        z = z + xT_ref[j] * vb[j:j + 1, :]
    deg = degp_ref[0] + degp_ref[1] + 2.0
    dinv = lax.rsqrt(deg)
    dinv_ref[...] = dinv
    zhat_ref[...] = dinv * z


def _tc_mid_body(pp_ref, zhat_ref, dinv_ref, b1_ref, W2_ref, Wl_ref, uhat_ref):
    c1 = jnp.dot(b1_ref[...], jnp.dot(W2_ref[...], Wl_ref[...]),
                 preferred_element_type=jnp.float32)
    dinv = dinv_ref[...]
    u = dinv * (pp_ref[0] + pp_ref[1] + 2.0 * zhat_ref[...]) + c1
    uhat_ref[...] = dinv * u


def _tc_post_body(qp_ref, uhat_ref, dinv_ref, b2_ref, Wl_ref, bl_ref, y_ref):
    c2 = jnp.dot(b2_ref[...], Wl_ref[...],
                 preferred_element_type=jnp.float32) + bl_ref[...]
    dinv = dinv_ref[...]
    w = dinv * (qp_ref[0] + qp_ref[1] + 2.0 * uhat_ref[...])
    y_ref[...] = w + c2


_vec_spec = pl.BlockSpec((BR, LANES), lambda i: (i, 0))
_part_spec = pl.BlockSpec((NCORE, BR, LANES), lambda i: (0, i, 0))
_vec_shape = jax.ShapeDtypeStruct((NROWS, LANES), jnp.float32)

_tc_pre = pl.pallas_call(
    _tc_pre_body,
    grid=(GRID,),
    in_specs=[
        pl.BlockSpec((H, BR, LANES), lambda i: (0, i, 0)),
        pl.BlockSpec((H, LANES), lambda i: (0, 0)),
        pl.BlockSpec((LANES, LANES), lambda i: (0, 0)),
        pl.BlockSpec((LANES, 1), lambda i: (0, 0)),
        _part_spec,
    ],
    out_specs=[_vec_spec, _vec_spec],
    out_shape=[_vec_shape, _vec_shape],
)

_tc_mid = pl.pallas_call(
    _tc_mid_body,
    grid=(GRID,),
    in_specs=[
        _part_spec,
        _vec_spec,
        _vec_spec,
        pl.BlockSpec((1, LANES), lambda i: (0, 0)),
        pl.BlockSpec((LANES, LANES), lambda i: (0, 0)),
        pl.BlockSpec((LANES, 1), lambda i: (0, 0)),
    ],
    out_specs=_vec_spec,
    out_shape=_vec_shape,
)

_tc_post = pl.pallas_call(
    _tc_post_body,
    grid=(GRID,),
    in_specs=[
        _part_spec,
        _vec_spec,
        _vec_spec,
        pl.BlockSpec((1, LANES), lambda i: (0, 0)),
        pl.BlockSpec((LANES, 1), lambda i: (0, 0)),
        pl.BlockSpec((1, 1), lambda i: (0, 0)),
    ],
    out_specs=_vec_spec,
    out_shape=_vec_shape,
)


def kernel(x, edge_index, W1, b1, W2, b2, Wl, bl):
    x = jnp.squeeze(x)
    ei = edge_index.astype(jnp.int32)
    xT = jnp.pad(x.T, ((0, 0), (0, NPAD - N))).reshape(H, NROWS, LANES)

    degp_raw, ei_flat = _sc_hist(ei)
    degp = degp_raw.reshape(NCORE, NROWS, LANES)
    dinv, zhat = _tc_pre(xT, W1, W2, Wl, degp)
    pp = _sc_spmv(ei_flat, zhat.reshape(NPAD)).reshape(NCORE, NROWS, LANES)
    c1 = b1 @ W2 @ Wl
    uhat = dinv * (dinv * (pp[0] + pp[1] + 2.0 * zhat) + c1[0])
    qp = _sc_spmv(ei_flat, uhat.reshape(NPAD)).reshape(NCORE, NROWS, LANES)
    c2 = b2 @ Wl + bl
    yv = dinv * (qp[0] + qp[1] + 2.0 * uhat) + c2[0]
    return yv.reshape(NPAD, 1)[:N]


# speedup vs baseline: 1.0302x; 1.0302x over previous
import jax
import jax.numpy as jnp
from jax import lax
from jax.experimental import pallas as pl
from jax.experimental.pallas import tpu as pltpu
from jax.experimental.pallas import tpu_sc as plsc

N = 100000
E = 1600000
H = 12
LANES = 128
NROWS = 800
NPAD = NROWS * LANES
NSUB = 16
NCORE = 2
NW = NSUB * NCORE
EPW = E // NW
NCH = 25
CH = EPW // NCH
NBLK = E // LANES
BPW = NBLK // NW
NCH_H = 26
BPC = BPW // NCH_H
CH_H = BPC * LANES
SL = NPAD // NSUB
GRID = 4
BR = NROWS // GRID

_sc_mesh = plsc.VectorSubcoreMesh(core_axis_name="c", subcore_axis_name="s")


def _zero_fill(ref, n):
    def body(i, carry):
        ref[pl.ds(i * 16, 16)] = jnp.zeros((16,), jnp.float32)
        return carry
    lax.fori_loop(0, n // 16, body, 0)


def _ones_fill(ref, n):
    def body(i, carry):
        ref[pl.ds(i * 16, 16)] = jnp.ones((16,), jnp.float32)
        return carry
    lax.fori_loop(0, n // 16, body, 0)


def _sc_hist_body(ei_h, out_h, flat_h, ibuf0, ibuf1, ibuf2,
                  didx0, didx1, didx2, vals, obuf, acc,
                  sem_i0, sem_i1, sem_i2, sem_s0, sem_s1, sem_s2,
                  sem_w0, sem_w1, sem_w2):
    c = lax.axis_index("c")
    s = lax.axis_index("s")
    w = c * NSUB + s
    _zero_fill(obuf, SL)
    _ones_fill(vals, CH_H)
    pltpu.sync_copy(obuf, acc.at[pl.ds(s * SL, SL)])
    plsc.subcore_barrier()
    base = w * BPW * LANES

    ibuf = [ibuf0, ibuf1, ibuf2]
    didx = [didx0, didx1, didx2]
    semi = [sem_i0, sem_i1, sem_i2]
    sems = [sem_s0, sem_s1, sem_s2]
    semw = [sem_w0, sem_w1, sem_w2]
    idesc = [None, None, None]
    sdesc = [None, None, None]
    wdesc = [None, None, None]

    def issue_idx(j):
        b = j % 3
        off = base + j * CH_H
        idesc[b] = pltpu.async_copy(
            ei_h.at[pl.ds(0, 2), pl.ds(off, CH_H)], ibuf[b], semi[b])

    issue_idx(0)
    issue_idx(1)
    for j in range(NCH_H):
        b = j % 3
        idesc[b].wait()
        idesc[b] = None
        off = base + j * CH_H

        @plsc.parallel_loop(0, CH_H // 16, unroll=5)
        def cp(i, _ib=ibuf[b], _db=didx[b]):
            _db[pl.ds(i * 16, 16)] = _ib[1, pl.ds(i * 16, 16)]

        sdesc[b] = pltpu.async_copy(vals, acc.at[didx[b]], sems[b],
                                    add=True)
        wdesc[b] = (
            pltpu.async_copy(ibuf[b].at[0], flat_h.at[pl.ds(off, CH_H)],
                             semw[b]),
            pltpu.async_copy(didx[b], flat_h.at[pl.ds(E + off, CH_H)],
                             semw[b]),
        )
        if j + 2 < NCH_H:
            nb = (j + 2) % 3
            if sdesc[nb] is not None:
                sdesc[nb].wait()
                sdesc[nb] = None
                for d in wdesc[nb]:
                    d.wait()
                wdesc[nb] = None
            issue_idx(j + 2)
    for b in range(3):
        if sdesc[b] is not None:
            sdesc[b].wait()
            for d in wdesc[b]:
                d.wait()

    @pl.when(w < 20)
    def _tail():
        off = (NW * BPW + w) * LANES
        pltpu.sync_copy(ei_h.at[pl.ds(0, 2), pl.ds(off, LANES)],
                        ibuf0.at[:, pl.ds(0, LANES)])

        @plsc.parallel_loop(0, LANES // 16, unroll=4)
        def cpt(i):
            didx0[pl.ds(i * 16, 16)] = ibuf0[1, pl.ds(i * 16, 16)]
            didx1[pl.ds(i * 16, 16)] = ibuf0[0, pl.ds(i * 16, 16)]

        pltpu.sync_copy(vals.at[pl.ds(0, LANES)],
                        acc.at[didx0.at[pl.ds(0, LANES)]], add=True)
        pltpu.sync_copy(didx1.at[pl.ds(0, LANES)],
                        flat_h.at[pl.ds(off, LANES)])
        pltpu.sync_copy(didx0.at[pl.ds(0, LANES)],
                        flat_h.at[pl.ds(E + off, LANES)])

    plsc.subcore_barrier()
    pltpu.sync_copy(acc.at[pl.ds(s * SL, SL)], obuf)
    pltpu.sync_copy(obuf, out_h.at[c, s])


_sc_hist = pl.kernel(
    _sc_hist_body,
    out_type=[
        jax.ShapeDtypeStruct((NCORE, NSUB, SL), jnp.float32),
        jax.ShapeDtypeStruct((2 * E,), jnp.int32),
    ],
    mesh=_sc_mesh,
    scratch_types=(
        [pltpu.VMEM((2, CH_H), jnp.int32) for _ in range(3)]
        + [pltpu.VMEM((CH_H,), jnp.int32) for _ in range(3)]
        + [
            pltpu.VMEM((CH_H,), jnp.float32),
            pltpu.VMEM((SL,), jnp.float32),
            pltpu.VMEM_SHARED((NPAD,), jnp.float32),
        ]
        + [pltpu.SemaphoreType.DMA for _ in range(9)]
    ),
)


def _sc_spmv_body(ei_h, vec_h, out_h, vtab,
                  sidx0, sidx1, sidx2, didx0, didx1, didx2,
                  vals0, vals1, vals2, acc,
                  sem_v, sem_i0, sem_i1, sem_i2, sem_s0, sem_s1, sem_s2):
    c = lax.axis_index("c")
    s = lax.axis_index("s")
    w = c * NSUB + s
    vdesc = pltpu.async_copy(vec_h.at[pl.ds(0, N)], vtab, sem_v)
    _zero_fill(vals0, CH)
    _zero_fill(vals1, CH)
    _zero_fill(vals2, CH)
    pltpu.sync_copy(vals0, acc.at[pl.ds(s * SL, CH)])
    pltpu.sync_copy(vals1, acc.at[pl.ds(s * SL + CH, CH)])
    pltpu.sync_copy(vals2, acc.at[pl.ds(s * SL + 2 * CH, CH)])
    pltpu.sync_copy(vals0.at[pl.ds(0, SL - 3 * CH)],
                    acc.at[pl.ds(s * SL + 3 * CH, SL - 3 * CH)])
    base = w * EPW

    sidx = [sidx0, sidx1, sidx2]
    didx = [didx0, didx1, didx2]
    vals = [vals0, vals1, vals2]
    semi = [sem_i0, sem_i1, sem_i2]
    sems = [sem_s0, sem_s1, sem_s2]
    idesc = [None, None, None]
    sdesc = [None, None, None]

    def issue_idx(j):
        b = j % 3
        off = base + j * CH
        idesc[b] = (
            pltpu.async_copy(ei_h.at[pl.ds(off, CH)], sidx[b], semi[b]),
            pltpu.async_copy(ei_h.at[pl.ds(E + off, CH)], didx[b], semi[b]),
        )

    issue_idx(0)
    issue_idx(1)
    vdesc.wait()
    plsc.subcore_barrier()
    for j in range(NCH):
        b = j % 3
        for d in idesc[b]:
            d.wait()
        idesc[b] = None

        @plsc.parallel_loop(0, CH // 16, unroll=5)
        def grp(i, _sb=sidx[b], _vb=vals[b]):
            si = _sb[pl.ds(i * 16, 16)]
            _vb[pl.ds(i * 16, 16)] = plsc.load_gather(vtab, [si])

        sdesc[b] = pltpu.async_copy(vals[b], acc.at[didx[b]], sems[b], add=True)
        if j + 2 < NCH:
            nb = (j + 2) % 3
            if sdesc[nb] is not None:
                sdesc[nb].wait()
                sdesc[nb] = None
            issue_idx(j + 2)
    for b in range(3):
        if sdesc[b] is not None:
            sdesc[b].wait()
    plsc.subcore_barrier()
    obase = c * NPAD + s * SL
    for k in range(3):
        pltpu.sync_copy(acc.at[pl.ds(s * SL + k * CH, CH)], vals[k])
        pltpu.sync_copy(vals[k], out_h.at[pl.ds(obase + k * CH, CH)])
    tail = SL - 3 * CH
    pltpu.sync_copy(acc.at[pl.ds(s * SL + 3 * CH, tail)], vals0.at[pl.ds(0, tail)])
    pltpu.sync_copy(vals0.at[pl.ds(0, tail)], out_h.at[pl.ds(obase + 3 * CH, tail)])


_sc_spmv = pl.kernel(
    _sc_spmv_body,
    out_type=jax.ShapeDtypeStruct((NCORE * NPAD,), jnp.float32),
    mesh=_sc_mesh,
    compiler_params=pltpu.CompilerParams(needs_layout_passes=False),
    scratch_types=(
        [pltpu.VMEM((N,), jnp.float32)]
        + [pltpu.VMEM((CH,), jnp.int32) for _ in range(6)]
        + [pltpu.VMEM((CH,), jnp.float32) for _ in range(3)]
        + [pltpu.VMEM_SHARED((NPAD,), jnp.float32)]
        + [pltpu.SemaphoreType.DMA for _ in range(7)]
    ),
)


def _tc_z_body(xT_ref, W1_ref, W2_ref, Wl_ref, z_ref):
    v = jnp.dot(W1_ref[...], jnp.dot(W2_ref[...], Wl_ref[...]),
                preferred_element_type=jnp.float32)
    vb = jnp.broadcast_to(v, (H, LANES))
    z = jnp.zeros((BR, LANES), jnp.float32)
    for j in range(H):
        z = z + xT_ref[j] * vb[j:j + 1, :]
    z_ref[...] = z


_vec_spec = pl.BlockSpec((BR, LANES), lambda i: (i, 0))
_part_spec = pl.BlockSpec((NCORE, BR, LANES), lambda i: (0, i, 0))
_vec_shape = jax.ShapeDtypeStruct((NROWS, LANES), jnp.float32)

_tc_z = pl.pallas_call(
    _tc_z_body,
    grid=(GRID,),
    in_specs=[
        pl.BlockSpec((H, BR, LANES), lambda i: (0, i, 0)),
        pl.BlockSpec((H, LANES), lambda i: (0, 0)),
        pl.BlockSpec((LANES, LANES), lambda i: (0, 0)),
        pl.BlockSpec((LANES, 1), lambda i: (0, 0)),
    ],
    out_specs=_vec_spec,
    out_shape=_vec_shape,
)

def kernel(x, edge_index, W1, b1, W2, b2, Wl, bl):
    x = jnp.squeeze(x)
    ei = edge_index.astype(jnp.int32)
    xT = jnp.pad(x.T, ((0, 0), (0, NPAD - N))).reshape(H, NROWS, LANES)

    z = _tc_z(xT, W1, W2, Wl)
    degp_raw, ei_flat = _sc_hist(ei)
    degp = degp_raw.reshape(NCORE, NROWS, LANES)
    dinv = lax.rsqrt(degp[0] + degp[1] + 2.0)
    zhat = dinv * z
    pp = _sc_spmv(ei_flat, zhat.reshape(NPAD)).reshape(NCORE, NROWS, LANES)
    c1 = b1 @ W2 @ Wl
    uhat = dinv * (dinv * (pp[0] + pp[1] + 2.0 * zhat) + c1[0])
    qp = _sc_spmv(ei_flat, uhat.reshape(NPAD)).reshape(NCORE, NROWS, LANES)
    c2 = b2 @ Wl + bl
    yv = dinv * (qp[0] + qp[1] + 2.0 * uhat) + c2[0]
    return yv.reshape(NPAD, 1)[:N]
